# baseline (speedup 1.0000x reference)
import functools

import jax
import jax.numpy as jnp
from jax import lax
from jax.experimental import pallas as pl
from jax.experimental.pallas import tpu as pltpu
from jax.experimental.pallas import tpu_sc as plsc

RB = 544
FC = 1024


def _router_body(x_ref, gate_ref,
                 logits_ref, xp_ref, pos0_ref, pos1_ref, rwb0_ref, rwb1_ref,
                 meta_ref, *, rb, nb):
    x = x_ref[...]
    s, d = x.shape
    n_e = gate_ref.shape[1]
    logits = lax.dot_general(x, gate_ref[...], (((1,), (0,)), ((), ())),
                             preferred_element_type=jnp.float32)
    logits_ref[...] = logits
    xlo = lax.bitcast_convert_type(
        x[:, :d // 2].astype(jnp.bfloat16).astype(jnp.float32), jnp.uint32)
    xhi = lax.bitcast_convert_type(
        x[:, d // 2:].astype(jnp.bfloat16).astype(jnp.float32), jnp.uint32)
    xp_ref[...] = lax.bitcast_convert_type(
        (xhi & jnp.uint32(0xFFFF0000)) | (xlo >> 16), jnp.int32)

    iota = lax.broadcasted_iota(jnp.int32, (s, n_e), 1)
    big = jnp.int32(2 ** 30)
    m1 = jnp.max(logits, axis=1, keepdims=True)
    i1 = jnp.min(jnp.where(logits == m1, iota, big), axis=1, keepdims=True)
    l2 = jnp.where(iota == i1, -jnp.inf, logits)
    m2 = jnp.max(l2, axis=1, keepdims=True)
    i2 = jnp.min(jnp.where(l2 == m2, iota, big), axis=1, keepdims=True)
    bexp = jnp.exp(m2 - m1)
    rw1 = 1.0 / (1.0 + bexp)
    rw2 = 1.0 - rw1
    rwb0_ref[...] = jnp.broadcast_to(rw1, (s, 128))
    rwb1_ref[...] = jnp.broadcast_to(rw2, (s, 128))

    m1f = (iota == i1).astype(jnp.float32)
    m2f = (iota == i2).astype(jnp.float32)

    ti = lax.broadcasted_iota(jnp.int32, (s, s), 0)
    tj = lax.broadcasted_iota(jnp.int32, (s, s), 1)
    lstrict = (tj < ti).astype(jnp.bfloat16)
    c1 = lax.dot_general(lstrict, m1f.astype(jnp.bfloat16),
                         (((1,), (0,)), ((), ())),
                         preferred_element_type=jnp.float32)
    c2 = lax.dot_general(lstrict, m2f.astype(jnp.bfloat16),
                         (((1,), (0,)), ((), ())),
                         preferred_element_type=jnp.float32)

    counts1 = jnp.sum(m1f, axis=0, keepdims=True)
    counts = counts1 + jnp.sum(m2f, axis=0, keepdims=True)
    rbf = jnp.float32(rb)
    pc = jnp.floor((counts + (rbf - 1.0)) / rbf) * rbf

    ei = lax.broadcasted_iota(jnp.int32, (n_e, n_e), 0)
    ej = lax.broadcasted_iota(jnp.int32, (n_e, n_e), 1)
    su = (ei < ej).astype(jnp.float32)
    offs = lax.dot_general(pc, su, (((1,), (0,)), ((), ())),
                           preferred_element_type=jnp.float32,
                           precision=lax.Precision.HIGHEST)

    rank0 = jnp.sum(m1f * c1, axis=1, keepdims=True)
    rank1 = jnp.sum(m2f * c2, axis=1, keepdims=True)
    off0 = jnp.sum(m1f * offs, axis=1, keepdims=True)
    off1 = jnp.sum(m2f * offs, axis=1, keepdims=True)
    c1sel = jnp.sum(m2f * counts1, axis=1, keepdims=True)
    pos0_ref[...] = (off0 + rank0).astype(jnp.int32)
    pos1_ref[...] = (off1 + c1sel + rank1).astype(jnp.int32)

    ones_col = jnp.ones((s, 1), jnp.bfloat16)
    counts_col = lax.dot_general((m1f + m2f).astype(jnp.bfloat16), ones_col,
                                 (((0,), (0,)), ((), ())),
                                 preferred_element_type=jnp.float32)
    pcc = jnp.floor((counts_col + (rbf - 1.0)) / rbf) * rbf
    sl = (ej < ei).astype(jnp.float32)
    offsc = lax.dot_general(sl, pcc, (((1,), (0,)), ((), ())),
                            preferred_element_type=jnp.float32,
                            precision=lax.Precision.HIGHEST)
    biota = lax.broadcasted_iota(jnp.int32, (1, 16), 1).astype(jnp.float32) * rbf
    bmask = (offsc <= biota).astype(jnp.float32)
    be = jnp.sum(bmask, axis=0, keepdims=True) - 1.0
    used = jnp.sum(pcc, axis=0, keepdims=True) / rbf
    meta = jnp.concatenate(
        [be, jnp.broadcast_to(used, (1, 16)),
         jnp.zeros((6, 16), jnp.float32)], axis=0)
    meta_ref[...] = meta.astype(jnp.int32)


def _group_mlp_body(meta_ref, xs_ref, w1_ref, w3_ref, w2_ref, wp_ref,
                    y_ref, xbf_ref, yacc_ref, *, n_f):
    b = pl.program_id(0)
    f = pl.program_id(1)
    used = meta_ref[16]

    @pl.when(b < used)
    def _():
        @pl.when(f == 0)
        def _():
            xi = lax.bitcast_convert_type(xs_ref[...], jnp.uint32)
            hw = xi.shape[1]
            xbf_ref[:, :hw] = lax.bitcast_convert_type(
                xi << 16, jnp.float32).astype(jnp.bfloat16)
            xbf_ref[:, hw:] = lax.bitcast_convert_type(
                xi & jnp.uint32(0xFFFF0000), jnp.float32).astype(jnp.bfloat16)

        xbf = xbf_ref[...]
        a1 = lax.dot_general(xbf, w1_ref[0], (((1,), (0,)), ((), ())),
                             preferred_element_type=jnp.float32)
        a3 = lax.dot_general(xbf, w3_ref[0], (((1,), (0,)), ((), ())),
                             preferred_element_type=jnp.float32)
        h = (a1 * lax.logistic(a1) * a3).astype(jnp.bfloat16)
        y = lax.dot_general(h, w2_ref[0], (((1,), (0,)), ((), ())),
                            preferred_element_type=jnp.float32)

        @pl.when(f == 0)
        def _():
            yacc_ref[...] = y

        @pl.when(f != 0)
        def _():
            yacc_ref[...] += y

        @pl.when(f == n_f - 1)
        def _():
            wcol = jnp.max(wp_ref[...], axis=1, keepdims=True)
            yt = yacc_ref[...] * wcol
            hw = yt.shape[1] // 2
            ylo = lax.bitcast_convert_type(
                yt[:, :hw].astype(jnp.bfloat16).astype(jnp.float32),
                jnp.uint32)
            yhi = lax.bitcast_convert_type(
                yt[:, hw:].astype(jnp.bfloat16).astype(jnp.float32),
                jnp.uint32)
            y_ref[...] = lax.bitcast_convert_type(
                (yhi & jnp.uint32(0xFFFF0000)) | (ylo >> 16), jnp.int32)


def kernel(hidden_states, gate_w, W1, W3, W2):
    b, s, d = hidden_states.shape
    n_e, _, f_dim = W1.shape
    n_f = f_dim // FC
    nb = (s * 2 + RB - 1) // RB + (n_e - 1)
    npad = nb * RB
    x = hidden_states.reshape(s, d)

    logits, xp, pos0, pos1, rwb0, rwb1, meta = pl.pallas_call(
        functools.partial(_router_body, rb=RB, nb=nb),
        in_specs=[pl.BlockSpec((s, d), lambda: (0, 0)),
                  pl.BlockSpec((d, n_e), lambda: (0, 0))],
        out_specs=[pl.BlockSpec((s, n_e), lambda: (0, 0)),
                   pl.BlockSpec((s, d // 2), lambda: (0, 0)),
                   pl.BlockSpec((s, 1), lambda: (0, 0)),
                   pl.BlockSpec((s, 1), lambda: (0, 0)),
                   pl.BlockSpec((s, 128), lambda: (0, 0)),
                   pl.BlockSpec((s, 128), lambda: (0, 0)),
                   pl.BlockSpec((8, 16), lambda: (0, 0))],
        out_shape=[jax.ShapeDtypeStruct((s, n_e), jnp.float32),
                   jax.ShapeDtypeStruct((s, d // 2), jnp.int32),
                   jax.ShapeDtypeStruct((s, 1), jnp.int32),
                   jax.ShapeDtypeStruct((s, 1), jnp.int32),
                   jax.ShapeDtypeStruct((s, 128), jnp.float32),
                   jax.ShapeDtypeStruct((s, 128), jnp.float32),
                   jax.ShapeDtypeStruct((8, 16), jnp.int32)],
    )(x, gate_w)

    p0 = pos0.reshape(s)
    p1 = pos1.reshape(s)

    try:
        info = plsc.get_sparse_core_info()
        nc, ns = info.num_cores, info.num_subcores
    except Exception:
        nc, ns = 2, 16
    nw = nc * ns
    tw = s // nw
    mesh = plsc.VectorSubcoreMesh(core_axis_name="c", subcore_axis_name="s")

    @functools.partial(
        pl.kernel,
        out_type=[jax.ShapeDtypeStruct((npad, d // 2), jnp.int32),
                  jax.ShapeDtypeStruct((npad, 128), jnp.float32)],
        mesh=mesh,
        scratch_types=[pltpu.VMEM((tw,), jnp.int32),
                       pltpu.VMEM((tw,), jnp.int32),
                       pltpu.VMEM((tw, d // 2), jnp.int32),
                       pltpu.VMEM((tw, 128), jnp.float32),
                       pltpu.VMEM((tw, 128), jnp.float32),
                       pltpu.SemaphoreType.DMA,
                       pltpu.SemaphoreType.DMA,
                       pltpu.SemaphoreType.DMA,
                       pltpu.SemaphoreType.DMA],
    )
    def _sc_dispatch(x_hbm, p0_hbm, p1_hbm, rwb0_hbm, rwb1_hbm,
                     xs_hbm, wp_hbm,
                     idx0_v, idx1_v, rows_v, w0_v, w1_v,
                     sem0, sem1, sem2, sem3):
        wid = lax.axis_index("s") * nc + lax.axis_index("c")
        base = wid * tw
        pltpu.sync_copy(p0_hbm.at[pl.ds(base, tw)], idx0_v)
        pltpu.sync_copy(p1_hbm.at[pl.ds(base, tw)], idx1_v)
        pltpu.sync_copy(x_hbm.at[pl.ds(base, tw)], rows_v)
        pltpu.sync_copy(rwb0_hbm.at[pl.ds(base, tw)], w0_v)
        pltpu.sync_copy(rwb1_hbm.at[pl.ds(base, tw)], w1_v)
        c0 = pltpu.async_copy(rows_v, xs_hbm.at[idx0_v], sem0)
        c1 = pltpu.async_copy(rows_v, xs_hbm.at[idx1_v], sem1)
        c2 = pltpu.async_copy(w0_v, wp_hbm.at[idx0_v], sem2)
        c3 = pltpu.async_copy(w1_v, wp_hbm.at[idx1_v], sem3)
        c0.wait()
        c1.wait()
        c2.wait()
        c3.wait()

    xs, wp = _sc_dispatch(xp, p0, p1, rwb0, rwb1)

    grid_spec = pltpu.PrefetchScalarGridSpec(
        num_scalar_prefetch=1,
        grid=(nb, n_f),
        in_specs=[
            pl.BlockSpec(
                (RB, d // 2),
                lambda bb, ff, m: (jnp.where(bb < m[16], bb, m[16] - 1), 0)),
            pl.BlockSpec(
                (1, d, FC),
                lambda bb, ff, m: (m[jnp.where(bb < m[16], bb, m[16] - 1)], 0,
                                   jnp.where(bb < m[16], ff, 0))),
            pl.BlockSpec(
                (1, d, FC),
                lambda bb, ff, m: (m[jnp.where(bb < m[16], bb, m[16] - 1)], 0,
                                   jnp.where(bb < m[16], ff, 0))),
            pl.BlockSpec(
                (1, FC, d),
                lambda bb, ff, m: (m[jnp.where(bb < m[16], bb, m[16] - 1)],
                                   jnp.where(bb < m[16], ff, 0), 0)),
            pl.BlockSpec(
                (RB, 128),
                lambda bb, ff, m: (jnp.where(bb < m[16], bb, m[16] - 1), 0)),
        ],
        out_specs=pl.BlockSpec(
            (RB, d // 2),
            lambda bb, ff, m: (jnp.where(bb < m[16], bb, m[16] - 1), 0)),
        scratch_shapes=[pltpu.VMEM((RB, d), jnp.bfloat16),
                        pltpu.VMEM((RB, d), jnp.float32)],
    )

    y = pl.pallas_call(
        functools.partial(_group_mlp_body, n_f=n_f),
        grid_spec=grid_spec,
        out_shape=jax.ShapeDtypeStruct((npad, d // 2), jnp.int32),
    )(meta.reshape(-1), xs, W1, W3, W2, wp)

    ct = 32
    nchunk = tw // ct

    @functools.partial(
        pl.kernel,
        out_type=jax.ShapeDtypeStruct((s, d), jnp.float32),
        mesh=mesh,
        scratch_types=[pltpu.VMEM((ct,), jnp.int32),
                       pltpu.VMEM((ct,), jnp.int32),
                       pltpu.VMEM((ct,), jnp.int32),
                       pltpu.VMEM((ct,), jnp.int32),
                       pltpu.VMEM((ct, d // 2), jnp.int32),
                       pltpu.VMEM((ct, d // 2), jnp.int32),
                       pltpu.VMEM((ct, d // 2), jnp.int32),
                       pltpu.VMEM((ct, d // 2), jnp.int32),
                       pltpu.VMEM((ct, d), jnp.float32),
                       pltpu.SemaphoreType.DMA,
                       pltpu.SemaphoreType.DMA,
                       pltpu.SemaphoreType.DMA,
                       pltpu.SemaphoreType.DMA],
    )
    def _sc_combine(y_hbm, p0_hbm, p1_hbm, out_hbm,
                    i0a, i1a, i0b, i1b, z0a, z1a, z0b, z1b, o_v,
                    s0a, s1a, s0b, s1b):
        wid = lax.axis_index("s") * nc + lax.axis_index("c")
        bufs = [(i0a, i1a, z0a, z1a, s0a, s1a),
                (i0b, i1b, z0b, z1b, s0b, s1b)]
        hw = d // 2
        himask = jnp.int32(-65536)

        def start(k, buf):
            i0, i1, z0, z1, sm0, sm1 = buf
            base = wid * tw + k * ct
            pltpu.sync_copy(p0_hbm.at[pl.ds(base, ct)], i0)
            pltpu.sync_copy(p1_hbm.at[pl.ds(base, ct)], i1)
            g0 = pltpu.async_copy(y_hbm.at[i0], z0, sm0)
            g1 = pltpu.async_copy(y_hbm.at[i1], z1, sm1)
            return g0, g1

        handles = [None, None]
        handles[0] = start(0, bufs[0])
        for k in range(nchunk):
            if k + 1 < nchunk:
                handles[(k + 1) % 2] = start(k + 1, bufs[(k + 1) % 2])
            g0, g1 = handles[k % 2]
            g0.wait()
            g1.wait()
            z0, z1 = bufs[k % 2][2], bufs[k % 2][3]

            def _row(r, _):
                def _col(c, _c):
                    for u in range(2):
                        sl = pl.ds(c * 32 + u * 16, 16)
                        slh = pl.ds(hw + c * 32 + u * 16, 16)
                        a = z0[r, sl]
                        bb2 = z1[r, sl]
                        o_v[r, sl] = (
                            lax.bitcast_convert_type(a << 16, jnp.float32)
                            + lax.bitcast_convert_type(bb2 << 16, jnp.float32))
                        o_v[r, slh] = (
                            lax.bitcast_convert_type(a & himask, jnp.float32)
                            + lax.bitcast_convert_type(bb2 & himask,
                                                       jnp.float32))
                    return _c
                return lax.fori_loop(0, hw // 32, _col, 0)

            lax.fori_loop(0, ct, _row, 0)
            pltpu.sync_copy(o_v, out_hbm.at[pl.ds(wid * tw + k * ct, ct)])

    out = _sc_combine(y, p0, p1)
    return out.reshape(b, s, d), logits.reshape(b, s, n_e)


# speedup vs baseline: 1.0199x; 1.0199x over previous
import functools

import jax
import jax.numpy as jnp
from jax import lax
from jax.experimental import pallas as pl
from jax.experimental.pallas import tpu as pltpu
from jax.experimental.pallas import tpu_sc as plsc

RB = 544
FC = 1024


def _router_body(x_ref, gate_ref,
                 logits_ref, xp_ref, pos0_ref, pos1_ref, rwb0_ref, rwb1_ref,
                 meta_ref, *, rb, nb):
    x = x_ref[...]
    s, d = x.shape
    n_e = gate_ref.shape[1]
    logits = lax.dot_general(x, gate_ref[...], (((1,), (0,)), ((), ())),
                             preferred_element_type=jnp.float32)
    logits_ref[...] = logits
    xlo = lax.bitcast_convert_type(
        x[:, :d // 2].astype(jnp.bfloat16).astype(jnp.float32), jnp.uint32)
    xhi = lax.bitcast_convert_type(
        x[:, d // 2:].astype(jnp.bfloat16).astype(jnp.float32), jnp.uint32)
    xp_ref[...] = lax.bitcast_convert_type(
        (xhi & jnp.uint32(0xFFFF0000)) | (xlo >> 16), jnp.int32)

    iota = lax.broadcasted_iota(jnp.int32, (s, n_e), 1)
    big = jnp.int32(2 ** 30)
    m1 = jnp.max(logits, axis=1, keepdims=True)
    i1 = jnp.min(jnp.where(logits == m1, iota, big), axis=1, keepdims=True)
    l2 = jnp.where(iota == i1, -jnp.inf, logits)
    m2 = jnp.max(l2, axis=1, keepdims=True)
    i2 = jnp.min(jnp.where(l2 == m2, iota, big), axis=1, keepdims=True)
    bexp = jnp.exp(m2 - m1)
    rw1 = 1.0 / (1.0 + bexp)
    rw2 = 1.0 - rw1
    rwb0_ref[...] = jnp.broadcast_to(rw1, (s, 128))
    rwb1_ref[...] = jnp.broadcast_to(rw2, (s, 128))

    m1f = (iota == i1).astype(jnp.float32)
    m2f = (iota == i2).astype(jnp.float32)

    ti = lax.broadcasted_iota(jnp.int32, (s, s), 0)
    tj = lax.broadcasted_iota(jnp.int32, (s, s), 1)
    lstrict = (tj < ti).astype(jnp.bfloat16)
    c1 = lax.dot_general(lstrict, m1f.astype(jnp.bfloat16),
                         (((1,), (0,)), ((), ())),
                         preferred_element_type=jnp.float32)
    c2 = lax.dot_general(lstrict, m2f.astype(jnp.bfloat16),
                         (((1,), (0,)), ((), ())),
                         preferred_element_type=jnp.float32)

    counts1 = jnp.sum(m1f, axis=0, keepdims=True)
    counts = counts1 + jnp.sum(m2f, axis=0, keepdims=True)
    rbf = jnp.float32(rb)
    pc = jnp.floor((counts + (rbf - 1.0)) / rbf) * rbf

    ei = lax.broadcasted_iota(jnp.int32, (n_e, n_e), 0)
    ej = lax.broadcasted_iota(jnp.int32, (n_e, n_e), 1)
    su = (ei < ej).astype(jnp.float32)
    offs = lax.dot_general(pc, su, (((1,), (0,)), ((), ())),
                           preferred_element_type=jnp.float32,
                           precision=lax.Precision.HIGHEST)

    rank0 = jnp.sum(m1f * c1, axis=1, keepdims=True)
    rank1 = jnp.sum(m2f * c2, axis=1, keepdims=True)
    off0 = jnp.sum(m1f * offs, axis=1, keepdims=True)
    off1 = jnp.sum(m2f * offs, axis=1, keepdims=True)
    c1sel = jnp.sum(m2f * counts1, axis=1, keepdims=True)
    pos0_ref[...] = (off0 + rank0).astype(jnp.int32)
    pos1_ref[...] = (off1 + c1sel + rank1).astype(jnp.int32)

    ones_col = jnp.ones((s, 1), jnp.bfloat16)
    counts_col = lax.dot_general((m1f + m2f).astype(jnp.bfloat16), ones_col,
                                 (((0,), (0,)), ((), ())),
                                 preferred_element_type=jnp.float32)
    pcc = jnp.floor((counts_col + (rbf - 1.0)) / rbf) * rbf
    sl = (ej < ei).astype(jnp.float32)
    offsc = lax.dot_general(sl, pcc, (((1,), (0,)), ((), ())),
                            preferred_element_type=jnp.float32,
                            precision=lax.Precision.HIGHEST)
    biota = lax.broadcasted_iota(jnp.int32, (1, 16), 1).astype(jnp.float32) * rbf
    bmask = (offsc <= biota).astype(jnp.float32)
    be = jnp.sum(bmask, axis=0, keepdims=True) - 1.0
    used = jnp.sum(pcc, axis=0, keepdims=True) / rbf
    meta = jnp.concatenate(
        [be, jnp.broadcast_to(used, (1, 16)),
         jnp.zeros((6, 16), jnp.float32)], axis=0)
    meta_ref[...] = meta.astype(jnp.int32)


def _group_mlp_body(meta_ref, xs_ref, w1_ref, w3_ref, w2_ref, wp_ref,
                    y_ref, xbf_ref, *, n_f):
    b = pl.program_id(0)
    f = pl.program_id(1)
    used = meta_ref[16]

    @pl.when(b < used)
    def _():
        @pl.when(f == 0)
        def _():
            xi = lax.bitcast_convert_type(xs_ref[...], jnp.uint32)
            hw = xi.shape[1]
            xbf_ref[:, :hw] = lax.bitcast_convert_type(
                xi << 16, jnp.float32).astype(jnp.bfloat16)
            xbf_ref[:, hw:] = lax.bitcast_convert_type(
                xi & jnp.uint32(0xFFFF0000), jnp.float32).astype(jnp.bfloat16)

        xbf = xbf_ref[...]
        a1 = lax.dot_general(xbf, w1_ref[0], (((1,), (0,)), ((), ())),
                             preferred_element_type=jnp.float32)
        a3 = lax.dot_general(xbf, w3_ref[0], (((1,), (0,)), ((), ())),
                             preferred_element_type=jnp.float32)
        h = (a1 * lax.logistic(a1) * a3).astype(jnp.bfloat16)
        y = lax.dot_general(h, w2_ref[0], (((1,), (0,)), ((), ())),
                            preferred_element_type=jnp.float32)

        @pl.when(f == 0)
        def _():
            y_ref[...] = y

        @pl.when(f != 0)
        def _():
            y_ref[...] += y

        @pl.when(f == n_f - 1)
        def _():
            wcol = jnp.max(wp_ref[...], axis=1, keepdims=True)
            y_ref[...] *= wcol


def kernel(hidden_states, gate_w, W1, W3, W2):
    b, s, d = hidden_states.shape
    n_e, _, f_dim = W1.shape
    n_f = f_dim // FC
    nb = (s * 2 + RB - 1) // RB + (n_e - 1)
    npad = nb * RB
    x = hidden_states.reshape(s, d)

    logits, xp, pos0, pos1, rwb0, rwb1, meta = pl.pallas_call(
        functools.partial(_router_body, rb=RB, nb=nb),
        in_specs=[pl.BlockSpec((s, d), lambda: (0, 0)),
                  pl.BlockSpec((d, n_e), lambda: (0, 0))],
        out_specs=[pl.BlockSpec((s, n_e), lambda: (0, 0)),
                   pl.BlockSpec((s, d // 2), lambda: (0, 0)),
                   pl.BlockSpec((s, 1), lambda: (0, 0)),
                   pl.BlockSpec((s, 1), lambda: (0, 0)),
                   pl.BlockSpec((s, 128), lambda: (0, 0)),
                   pl.BlockSpec((s, 128), lambda: (0, 0)),
                   pl.BlockSpec((8, 16), lambda: (0, 0))],
        out_shape=[jax.ShapeDtypeStruct((s, n_e), jnp.float32),
                   jax.ShapeDtypeStruct((s, d // 2), jnp.int32),
                   jax.ShapeDtypeStruct((s, 1), jnp.int32),
                   jax.ShapeDtypeStruct((s, 1), jnp.int32),
                   jax.ShapeDtypeStruct((s, 128), jnp.float32),
                   jax.ShapeDtypeStruct((s, 128), jnp.float32),
                   jax.ShapeDtypeStruct((8, 16), jnp.int32)],
    )(x, gate_w)

    p0 = pos0.reshape(s)
    p1 = pos1.reshape(s)

    try:
        info = plsc.get_sparse_core_info()
        nc, ns = info.num_cores, info.num_subcores
    except Exception:
        nc, ns = 2, 16
    nw = nc * ns
    tw = s // nw
    mesh = plsc.VectorSubcoreMesh(core_axis_name="c", subcore_axis_name="s")

    @functools.partial(
        pl.kernel,
        out_type=[jax.ShapeDtypeStruct((npad, d // 2), jnp.int32),
                  jax.ShapeDtypeStruct((npad, 128), jnp.float32)],
        mesh=mesh,
        scratch_types=[pltpu.VMEM((tw,), jnp.int32),
                       pltpu.VMEM((tw,), jnp.int32),
                       pltpu.VMEM((tw, d // 2), jnp.int32),
                       pltpu.VMEM((tw, 128), jnp.float32),
                       pltpu.VMEM((tw, 128), jnp.float32),
                       pltpu.SemaphoreType.DMA,
                       pltpu.SemaphoreType.DMA,
                       pltpu.SemaphoreType.DMA,
                       pltpu.SemaphoreType.DMA],
    )
    def _sc_dispatch(x_hbm, p0_hbm, p1_hbm, rwb0_hbm, rwb1_hbm,
                     xs_hbm, wp_hbm,
                     idx0_v, idx1_v, rows_v, w0_v, w1_v,
                     sem0, sem1, sem2, sem3):
        wid = lax.axis_index("s") * nc + lax.axis_index("c")
        base = wid * tw
        pltpu.sync_copy(p0_hbm.at[pl.ds(base, tw)], idx0_v)
        pltpu.sync_copy(p1_hbm.at[pl.ds(base, tw)], idx1_v)
        pltpu.sync_copy(x_hbm.at[pl.ds(base, tw)], rows_v)
        pltpu.sync_copy(rwb0_hbm.at[pl.ds(base, tw)], w0_v)
        pltpu.sync_copy(rwb1_hbm.at[pl.ds(base, tw)], w1_v)
        c0 = pltpu.async_copy(rows_v, xs_hbm.at[idx0_v], sem0)
        c1 = pltpu.async_copy(rows_v, xs_hbm.at[idx1_v], sem1)
        c2 = pltpu.async_copy(w0_v, wp_hbm.at[idx0_v], sem2)
        c3 = pltpu.async_copy(w1_v, wp_hbm.at[idx1_v], sem3)
        c0.wait()
        c1.wait()
        c2.wait()
        c3.wait()

    xs, wp = _sc_dispatch(xp, p0, p1, rwb0, rwb1)

    grid_spec = pltpu.PrefetchScalarGridSpec(
        num_scalar_prefetch=1,
        grid=(nb, n_f),
        in_specs=[
            pl.BlockSpec(
                (RB, d // 2),
                lambda bb, ff, m: (jnp.where(bb < m[16], bb, m[16] - 1), 0)),
            pl.BlockSpec(
                (1, d, FC),
                lambda bb, ff, m: (m[jnp.where(bb < m[16], bb, m[16] - 1)], 0,
                                   jnp.where(bb < m[16], ff, 0))),
            pl.BlockSpec(
                (1, d, FC),
                lambda bb, ff, m: (m[jnp.where(bb < m[16], bb, m[16] - 1)], 0,
                                   jnp.where(bb < m[16], ff, 0))),
            pl.BlockSpec(
                (1, FC, d),
                lambda bb, ff, m: (m[jnp.where(bb < m[16], bb, m[16] - 1)],
                                   jnp.where(bb < m[16], ff, 0), 0)),
            pl.BlockSpec(
                (RB, 128),
                lambda bb, ff, m: (jnp.where(bb < m[16], bb, m[16] - 1), 0)),
        ],
        out_specs=pl.BlockSpec(
            (RB, d),
            lambda bb, ff, m: (jnp.where(bb < m[16], bb, m[16] - 1), 0)),
        scratch_shapes=[pltpu.VMEM((RB, d), jnp.bfloat16)],
    )

    y = pl.pallas_call(
        functools.partial(_group_mlp_body, n_f=n_f),
        grid_spec=grid_spec,
        out_shape=jax.ShapeDtypeStruct((npad, d), jnp.float32),
    )(meta.reshape(-1), xs, W1, W3, W2, wp)

    ct = 16
    nchunk = tw // ct

    @functools.partial(
        pl.kernel,
        out_type=jax.ShapeDtypeStruct((s, d), jnp.float32),
        mesh=mesh,
        scratch_types=[pltpu.VMEM((ct,), jnp.int32),
                       pltpu.VMEM((ct,), jnp.int32),
                       pltpu.VMEM((ct,), jnp.int32),
                       pltpu.VMEM((ct,), jnp.int32),
                       pltpu.VMEM((ct, d), jnp.float32),
                       pltpu.VMEM((ct, d), jnp.float32),
                       pltpu.VMEM((ct, d), jnp.float32),
                       pltpu.VMEM((ct, d), jnp.float32),
                       pltpu.SemaphoreType.DMA,
                       pltpu.SemaphoreType.DMA,
                       pltpu.SemaphoreType.DMA,
                       pltpu.SemaphoreType.DMA],
    )
    def _sc_combine(y_hbm, p0_hbm, p1_hbm, out_hbm,
                    i0a, i1a, i0b, i1b, z0a, z1a, z0b, z1b,
                    s0a, s1a, s0b, s1b):
        wid = lax.axis_index("s") * nc + lax.axis_index("c")
        bufs = [(i0a, i1a, z0a, z1a, s0a, s1a),
                (i0b, i1b, z0b, z1b, s0b, s1b)]

        def start(k, buf):
            i0, i1, z0, z1, sm0, sm1 = buf
            base = wid * tw + k * ct
            pltpu.sync_copy(p0_hbm.at[pl.ds(base, ct)], i0)
            pltpu.sync_copy(p1_hbm.at[pl.ds(base, ct)], i1)
            g0 = pltpu.async_copy(y_hbm.at[i0], z0, sm0)
            g1 = pltpu.async_copy(y_hbm.at[i1], z1, sm1)
            return g0, g1

        handles = [None, None]
        handles[0] = start(0, bufs[0])
        for k in range(nchunk):
            if k + 1 < nchunk:
                handles[(k + 1) % 2] = start(k + 1, bufs[(k + 1) % 2])
            g0, g1 = handles[k % 2]
            g0.wait()
            g1.wait()
            z0, z1 = bufs[k % 2][2], bufs[k % 2][3]

            def _row(r, _):
                def _col(c, _c):
                    for u in range(4):
                        sl = pl.ds(c * 64 + u * 16, 16)
                        z0[r, sl] = z0[r, sl] + z1[r, sl]
                    return _c
                return lax.fori_loop(0, d // 64, _col, 0)

            lax.fori_loop(0, ct, _row, 0)
            pltpu.sync_copy(z0, out_hbm.at[pl.ds(wid * tw + k * ct, ct)])

    out = _sc_combine(y, p0, p1)
    return out.reshape(b, s, d), logits.reshape(b, s, n_e)
